# baseline (speedup 1.0000x reference)
import functools

import jax
import jax.numpy as jnp
import numpy as np
from jax import lax
from jax.experimental import pallas as pl
from jax.experimental.pallas import tpu as pltpu
from jax.experimental.pallas import tpu_sc as plsc

B, N, D, K = 32, 576, 768, 432
U = N - K
CHUNK = 72
NMC = K // CHUNK
NUC = U // CHUNK


@functools.lru_cache(maxsize=1)
def _constants():
    with jax.ensure_compile_time_eval():
        u = jax.random.uniform(jax.random.key(42), (B, N))
        idx = np.asarray(jax.device_get(jnp.argsort(u, axis=-1)))
    base = np.arange(B, dtype=np.int64)[:, None] * N
    midx = (base + idx[:, :K]).reshape(B, NMC, CHUNK).astype(np.int32)
    uidx = (base + np.sort(idx[:, K:], axis=-1)
            ).reshape(B, NUC, CHUNK).astype(np.int32)
    lidx = idx[:, :K].astype(np.int32).reshape(B, 1, K)
    return midx, uidx, lidx


def _images_kernel(flat_patches, tokens, midx, uidx):
    info = plsc.get_sparse_core_info()
    nc = info.num_cores

    @functools.partial(
        pl.kernel,
        mesh=plsc.VectorSubcoreMesh(core_axis_name="c", subcore_axis_name="s"),
        out_type=jax.ShapeDtypeStruct((B * N, D), jnp.float32),
        scratch_types=[
            pltpu.VMEM((NMC, CHUNK), jnp.int32),
            pltpu.VMEM((NUC, CHUNK), jnp.int32),
            pltpu.VMEM((CHUNK, D), jnp.float32),
            pltpu.VMEM((CHUNK, D), jnp.float32),
            pltpu.SemaphoreType.DMA,
            pltpu.SemaphoreType.DMA,
            pltpu.SemaphoreType.DMA,
            pltpu.SemaphoreType.DMA,
        ],
    )
    def k(patches_hbm, tokens_hbm, midx_hbm, uidx_hbm, images_hbm,
          midx_v, uidx_v, tok_v, buf, gsem, wsem, tsem, fsem):
        wid = lax.axis_index("s") * nc + lax.axis_index("c")
        pltpu.sync_copy(midx_hbm.at[wid], midx_v)
        pltpu.sync_copy(uidx_hbm.at[wid], uidx_v)
        fill = pltpu.async_copy(tokens_hbm, tok_v, fsem)
        g = pltpu.async_copy(patches_hbm.at[uidx_v.at[0]], buf, gsem)
        fill.wait()
        tsc = [pltpu.async_copy(tok_v, images_hbm.at[midx_v.at[j]], tsem)
               for j in range(NMC)]
        for j in range(NUC):
            g.wait()
            pltpu.async_copy(buf, images_hbm.at[uidx_v.at[j]], wsem).wait()
            if j + 1 < NUC:
                g = pltpu.async_copy(
                    patches_hbm.at[uidx_v.at[j + 1]], buf, gsem)
        for c in tsc:
            c.wait()

    return k(flat_patches, tokens, midx, uidx)


def _mp_body(lidx_ref, patches_ref, out_ref):
    sel = lidx_ref[0, 0, :]
    onehot = (lax.broadcasted_iota(jnp.int32, (K, N), 1)
              == sel[:, None]).astype(jnp.float32)
    out_ref[0] = jnp.dot(onehot, patches_ref[0],
                         preferred_element_type=jnp.float32)


def _mp_kernel(patches, lidx):
    return pl.pallas_call(
        _mp_body,
        grid=(B,),
        in_specs=[
            pl.BlockSpec((1, 1, K), lambda b: (b, 0, 0)),
            pl.BlockSpec((1, N, D), lambda b: (b, 0, 0)),
        ],
        out_specs=pl.BlockSpec((1, K, D), lambda b: (b, 0, 0)),
        out_shape=jax.ShapeDtypeStruct((B, K, D), jnp.float32),
    )(lidx, patches)


def kernel(patches, mask):
    midx_np, uidx_np, lidx_np = _constants()
    midx = jnp.asarray(midx_np)
    uidx = jnp.asarray(uidx_np)
    lidx = jnp.asarray(lidx_np)
    tokens = jnp.broadcast_to(mask, (CHUNK, D))
    flat = patches.reshape(B * N, D)
    images = _images_kernel(flat, tokens, midx, uidx)
    masked_patches = _mp_kernel(patches, lidx)
    return (images.reshape(B, N, D), masked_patches)
